# baseline (speedup 1.0000x reference)
import jax
import jax.numpy as jnp
from jax.experimental import pallas as pl
from jax.experimental.pallas import tpu as pltpu

B, W, D, H, OUT = 512, 5, 256, 128, 10


def _gru(h, wl, bi, bh):
    g = jax.lax.dot_general(
        h.astype(jnp.bfloat16), wl, (((1,), (1,)), ((), ())),
        preferred_element_type=jnp.float32
    ) + bi
    t = jnp.tanh(0.5 * (g[:, :4 * H] + bh[:, :4 * H]))
    r = 0.5 + 0.5 * t[:, :2 * H]
    zc = 0.5 - 0.5 * t[:, 2 * H:]
    n = jnp.tanh(g[:, 4 * H:] + r * bh[:, 4 * H:])
    return zc * n


def _fused_kernel(x_ref, wg_ref, bb_ref, g1_ref, g2_ref, fw_ref, out_ref):
    xt = x_ref[:]
    out1 = _gru(xt, wg_ref[:6 * H, :], bb_ref[0:1, :], bb_ref[1:2, :])
    out2 = _gru(out1, wg_ref[6 * H:, :], bb_ref[2:3, :], bb_ref[3:4, :])
    m = jnp.sum(out2, axis=0, keepdims=True) * (1.0 / B)
    v1 = jnp.dot(m, g1_ref[:],
                 preferred_element_type=jnp.float32) + bb_ref[4:5, :H]
    v2 = jnp.dot(v1, g2_ref[:],
                 preferred_element_type=jnp.float32) + bb_ref[5:6, :64]
    o = jax.lax.dot_general(
        v2, fw_ref[:], (((1,), (1,)), ((), ())),
        preferred_element_type=jnp.float32) + bb_ref[6:7, :OUT]
    out_ref[:] = jnp.broadcast_to(o, (B, OUT))


def _pack_w(wf, wr):
    return jnp.concatenate(
        [wf[:H], wr[:H], wf[H:2 * H], wr[H:2 * H], wf[2 * H:], wr[2 * H:]],
        axis=0)


def _pack_b(bf, br):
    return jnp.concatenate(
        [bf[:H], br[:H], bf[H:2 * H], br[H:2 * H], bf[2 * H:], br[2 * H:]])


def _pad_row(v):
    return jnp.pad(v, (0, 6 * H - v.shape[0]))


@jax.jit
def kernel(x, g1_wih_f, g1_bih_f, g1_bhh_f, g1_wih_r, g1_bih_r, g1_bhh_r,
           g2_wih_f, g2_bih_f, g2_bhh_f, g2_wih_r, g2_bih_r, g2_bhh_r,
           gcn1_w, gcn1_b, gcn2_w, gcn2_b, fc_w, fc_b):
    xf = x.reshape(B, W * D)
    wg = jnp.concatenate([_pack_w(g1_wih_f, g1_wih_r),
                          _pack_w(g2_wih_f, g2_wih_r)],
                         axis=0).astype(jnp.bfloat16)
    bb = jnp.stack([_pack_b(g1_bih_f, g1_bih_r),
                    _pack_b(g1_bhh_f, g1_bhh_r),
                    _pack_b(g2_bih_f, g2_bih_r),
                    _pack_b(g2_bhh_f, g2_bhh_r),
                    _pad_row(gcn1_b),
                    _pad_row(gcn2_b),
                    _pad_row(fc_b),
                    jnp.zeros((6 * H,), jnp.float32)])
    return pl.pallas_call(
        _fused_kernel,
        grid=(1,),
        out_shape=jax.ShapeDtypeStruct((B, OUT), jnp.float32),
        in_specs=[pl.BlockSpec((B, D), lambda i: (0, W - 1))]
        + [pl.BlockSpec(memory_space=pltpu.VMEM)] * 5,
        out_specs=pl.BlockSpec(memory_space=pltpu.VMEM),
    )(xf, wg, bb, gcn1_w, gcn2_w, fc_w)
